# baseline (speedup 1.0000x reference)
import functools
import math

import jax
import jax.numpy as jnp
from jax import lax
from jax.experimental import pallas as pl
from jax.experimental.pallas import tpu as pltpu
from jax.experimental.pallas import tpu_sc as plsc

D_MODEL = 64
SCALE = math.sqrt(D_MODEL)
NUM_CORES = 2
NUM_SUBCORES = 16
NUM_WORKERS = NUM_CORES * NUM_SUBCORES
CHUNK = 512


@functools.partial(jax.jit, static_argnames=("batch",))
def _embed(xf, lut, batch):
    b_per_w = batch // NUM_WORKERS
    n_chunks = b_per_w // CHUNK
    mesh = plsc.VectorSubcoreMesh(core_axis_name="c", subcore_axis_name="s")

    @functools.partial(
        pl.kernel,
        mesh=mesh,
        out_type=jax.ShapeDtypeStruct((batch, D_MODEL), jnp.float32),
        scratch_types=[
            pltpu.VMEM((CHUNK,), jnp.int32),
            pltpu.VMEM((CHUNK, D_MODEL), jnp.float32),
            pltpu.SemaphoreType.DMA,
        ],
        compiler_params=pltpu.CompilerParams(use_tc_tiling_on_sc=False),
    )
    def emb(idx_hbm, tab_hbm, out_hbm, idx_v, rows_v, sem):
        wid = lax.axis_index("s") * NUM_CORES + lax.axis_index("c")
        base = wid * b_per_w

        def chunk_body(j, carry):
            off = base + j * CHUNK
            pltpu.sync_copy(idx_hbm.at[pl.ds(off, CHUNK)], idx_v)
            pltpu.async_copy(tab_hbm.at[idx_v], rows_v, sem).wait()

            def scale_body(i, c):
                for t in range(D_MODEL // 16):
                    sl = pl.ds(t * 16, 16)
                    rows_v[i, sl] = rows_v[i, sl] * SCALE
                return c

            lax.fori_loop(0, CHUNK, scale_body, 0)
            pltpu.sync_copy(rows_v, out_hbm.at[pl.ds(off, CHUNK)])
            return carry

        lax.fori_loop(0, n_chunks, chunk_body, 0)

    return emb(xf, lut)


def kernel(x, lut):
    batch = x.size
    xf = x.reshape(batch).astype(jnp.int32)
    out = _embed(xf, lut, batch)
    return out.reshape(x.shape + (D_MODEL,))


# speedup vs baseline: 1.1339x; 1.1339x over previous
import functools
import math

import jax
import jax.numpy as jnp
from jax import lax
from jax.experimental import pallas as pl
from jax.experimental.pallas import tpu as pltpu
from jax.experimental.pallas import tpu_sc as plsc

D_MODEL = 64
SCALE = math.sqrt(D_MODEL)
NUM_CORES = 2
NUM_SUBCORES = 16
NUM_WORKERS = NUM_CORES * NUM_SUBCORES
CHUNK = 512
NBUF = 3


@functools.partial(jax.jit, static_argnames=("batch",))
def _embed(xf, lut, batch):
    b_per_w = batch // NUM_WORKERS
    n_chunks = b_per_w // CHUNK
    n_main = (n_chunks // NBUF) * NBUF
    mesh = plsc.VectorSubcoreMesh(core_axis_name="c", subcore_axis_name="s")

    @functools.partial(
        pl.kernel,
        mesh=mesh,
        out_type=jax.ShapeDtypeStruct((batch, D_MODEL), jnp.float32),
        scratch_types=[
            pltpu.VMEM((b_per_w,), jnp.int32),
            pltpu.VMEM((NBUF, CHUNK, D_MODEL), jnp.float32),
            [pltpu.SemaphoreType.DMA] * NBUF,
            [pltpu.SemaphoreType.DMA] * NBUF,
        ],
        compiler_params=pltpu.CompilerParams(use_tc_tiling_on_sc=False),
    )
    def emb(idx_hbm, tab_hbm, out_hbm, idx_v, rows_v, gsem, ssem):
        wid = lax.axis_index("s") * NUM_CORES + lax.axis_index("c")
        base = wid * b_per_w

        pltpu.sync_copy(idx_hbm.at[pl.ds(base, b_per_w)], idx_v)

        def start_gather(chunk, b):
            idx_slice = idx_v.at[pl.ds(chunk * CHUNK, CHUNK)]
            pltpu.async_copy(tab_hbm.at[idx_slice], rows_v.at[b], gsem[b])

        def wait_gather(chunk, b):
            idx_slice = idx_v.at[pl.ds(chunk * CHUNK, CHUNK)]
            pltpu.make_async_copy(
                tab_hbm.at[idx_slice], rows_v.at[b], gsem[b]
            ).wait()

        def scale(b):
            def srow(i, c):
                for r in range(4):
                    for t in range(D_MODEL // 16):
                        sl = pl.ds(t * 16, 16)
                        rows_v[b, i * 4 + r, sl] = rows_v[b, i * 4 + r, sl] * SCALE
                return c

            lax.fori_loop(0, CHUNK // 4, srow, 0)

        def out_slice(chunk):
            return out_hbm.at[pl.ds(base + chunk * CHUNK, CHUNK)]

        def start_store(chunk, b):
            pltpu.async_copy(rows_v.at[b], out_slice(chunk), ssem[b])

        def wait_store(chunk, b):
            pltpu.make_async_copy(rows_v.at[b], out_slice(chunk), ssem[b]).wait()

        start_gather(0, 0)
        start_gather(1, 1)

        def step(chunk, b):
            wait_gather(chunk, b)
            scale(b)
            start_store(chunk, b)
            nb = (b + 2) % NBUF

            @pl.when(chunk >= 1)
            def _():
                wait_store(chunk - 1, nb)

            start_gather(chunk + 2, nb)

        def main_body(s, c):
            for u in range(NBUF):
                step(s * NBUF + u, u)
            return c

        lax.fori_loop(0, n_main // NBUF, main_body, 0)

        for chunk in range(n_main, n_chunks):
            b = chunk % NBUF
            wait_gather(chunk, b)
            scale(b)
            start_store(chunk, b)

        for chunk in range(n_chunks - NBUF, n_chunks):
            wait_store(chunk, chunk % NBUF)

    return emb(xf, lut)


def kernel(x, lut):
    batch = x.size
    xf = x.reshape(batch).astype(jnp.int32)
    out = _embed(xf, lut, batch)
    return out.reshape(x.shape + (D_MODEL,))
